# baseline (speedup 1.0000x reference)
import jax
import jax.numpy as jnp
from jax.experimental import pallas as pl
from jax.experimental.pallas import tpu as pltpu

_KEEP_RATIO = 0.1
_B = 64
_H = 512
_W = 512
_N = _H * _W
_K = max(1, int(_N * _KEEP_RATIO))
_S = 4
_ITERS = 31


def _bce_with_logits(logits, targets):
    return (jnp.maximum(logits, 0.0) - logits * targets
            + jnp.log1p(jnp.exp(-jnp.abs(logits))))


def _topk_kernel(logits_ref, targets_ref, out_ref, key_ref):
    step = pl.program_id(0)
    x = logits_ref[:, 0, :, :]
    t = targets_ref[:, 0, :, :]
    loss = jnp.maximum(_bce_with_logits(x, t), 0.0)
    key_ref[...] = jax.lax.bitcast_convert_type(loss, jnp.int32)

    keys = key_ref[...]
    kcount = jnp.int32(_K)
    lo0 = jnp.full((_S, 1, 1), -1, jnp.int32)
    hi0 = jnp.full((_S, 1, 1), 0x7F800000, jnp.int32)

    def body(_, carry):
        lo, hi = carry
        mid = lo + jax.lax.shift_right_logical(hi - lo, 1)
        cnt = jnp.sum((keys > mid).astype(jnp.int32), axis=(1, 2),
                      keepdims=True)
        keep_lo = cnt >= kcount
        lo = jnp.where(keep_lo, mid, lo)
        hi = jnp.where(keep_lo, hi, mid)
        return lo, hi

    _, tkey = jax.lax.fori_loop(0, _ITERS, body, (lo0, hi0))

    gt = keys > tkey
    cnt_gt = jnp.sum(gt.astype(jnp.float32), axis=(1, 2), keepdims=True)
    vals = jax.lax.bitcast_convert_type(keys, jnp.float32)
    sum_gt = jnp.sum(jnp.where(gt, vals, 0.0), axis=(1, 2), keepdims=True)
    tval = jax.lax.bitcast_convert_type(tkey, jnp.float32)
    part = jnp.sum(sum_gt + tval * (jnp.float32(_K) - cnt_gt))
    del step
    out_ref[...] = jnp.reshape(part, (1, 1, 1))


def kernel(logits, targets):
    out = pl.pallas_call(
        _topk_kernel,
        grid=(_B // _S,),
        in_specs=[
            pl.BlockSpec((_S, 1, _H, _W), lambda b: (b, 0, 0, 0)),
            pl.BlockSpec((_S, 1, _H, _W), lambda b: (b, 0, 0, 0)),
        ],
        out_specs=pl.BlockSpec((1, 1, 1), lambda b: (b, 0, 0)),
        out_shape=jax.ShapeDtypeStruct((_B // _S, 1, 1), jnp.float32),
        scratch_shapes=[pltpu.VMEM((_S, _H, _W), jnp.int32)],
        compiler_params=pltpu.CompilerParams(
            dimension_semantics=("parallel",)),
    )(logits, targets)
    return jnp.sum(out) / jnp.float32(_B * _K)


# speedup vs baseline: 1.3732x; 1.3732x over previous
import jax
import jax.numpy as jnp
from jax.experimental import pallas as pl
from jax.experimental.pallas import tpu as pltpu

_KEEP_RATIO = 0.1
_B = 64
_H = 512
_W = 512
_N = _H * _W
_K = max(1, int(_N * _KEEP_RATIO))
_S = 4
_SHIFT = 12
_ITERS = 20


def _bce_with_logits(logits, targets):
    return (jnp.maximum(logits, 0.0) - logits * targets
            + jnp.log1p(jnp.exp(-jnp.abs(logits))))


def _topk_kernel(logits_ref, targets_ref, out_ref, loss_ref):
    x = logits_ref[:, 0, :, :]
    t = targets_ref[:, 0, :, :]
    loss_ref[...] = jnp.maximum(_bce_with_logits(x, t), 0.0)

    loss = loss_ref[...]
    kcount = jnp.int32(_K)
    lo0 = jnp.full((_S, 1, 1), -1, jnp.int32)
    hi0 = jnp.full((_S, 1, 1), 0x7F800000 >> _SHIFT, jnp.int32)

    def _upper_edge(code):
        return jax.lax.bitcast_convert_type((code + 1) << _SHIFT, jnp.float32)

    def body(_, carry):
        lo, hi = carry
        mid = lo + jax.lax.shift_right_logical(hi - lo, 1)
        cnt = jnp.sum((loss >= _upper_edge(mid)).astype(jnp.int32),
                      axis=(1, 2), keepdims=True)
        keep_lo = cnt >= kcount
        lo = jnp.where(keep_lo, mid, lo)
        hi = jnp.where(keep_lo, hi, mid)
        return lo, hi

    _, tcode = jax.lax.fori_loop(0, _ITERS, body, (lo0, hi0))

    ub = _upper_edge(tcode)
    ge = loss >= ub
    cnt_ge = jnp.sum(ge.astype(jnp.float32), axis=(1, 2), keepdims=True)
    sum_ge = jnp.sum(jnp.where(ge, loss, 0.0), axis=(1, 2), keepdims=True)
    tval = jax.lax.bitcast_convert_type(tcode << _SHIFT, jnp.float32)
    part = jnp.sum(sum_ge + tval * (jnp.float32(_K) - cnt_ge))
    out_ref[...] = jnp.reshape(part, (1, 1, 1))


def kernel(logits, targets):
    out = pl.pallas_call(
        _topk_kernel,
        grid=(_B // _S,),
        in_specs=[
            pl.BlockSpec((_S, 1, _H, _W), lambda b: (b, 0, 0, 0)),
            pl.BlockSpec((_S, 1, _H, _W), lambda b: (b, 0, 0, 0)),
        ],
        out_specs=pl.BlockSpec((1, 1, 1), lambda b: (b, 0, 0)),
        out_shape=jax.ShapeDtypeStruct((_B // _S, 1, 1), jnp.float32),
        scratch_shapes=[pltpu.VMEM((_S, _H, _W), jnp.float32)],
        compiler_params=pltpu.CompilerParams(
            dimension_semantics=("arbitrary",)),
    )(logits, targets)
    return jnp.sum(out) / jnp.float32(_B * _K)
